# Initial kernel scaffold; baseline (speedup 1.0000x reference)
import jax
import jax.numpy as jnp
from jax.experimental import pallas as pl


def kernel(ent_emb, rel_emb, edge_index, rel_id, neigh_w, bn_gamma, bn_beta):
    raise NotImplementedError("write your pallas kernel here")


# speedup vs baseline: 34.7045x; 34.7045x over previous
import functools

import jax
import jax.numpy as jnp
from jax import lax
from jax.experimental import pallas as pl
from jax.experimental.pallas import tpu as pltpu
from jax.experimental.pallas import tpu_sc as plsc

N_ENT = 10000
N_REL2 = 400
EMB = 128
N_EDGES = 320000

NUM_SC = 2
NUM_TILES = 16
HALF = N_ENT // NUM_SC
HIST_WORDS = HALF * N_REL2
ZSPAN = HIST_WORDS // NUM_TILES
E_PER_TILE = 20480
CHUNK = 1024
N_CHUNKS = E_PER_TILE // CHUNK
E_PAD = E_PER_TILE * NUM_TILES


def _hist_body(dst_hbm, rel_hbm, out_hbm, hist_sh, dst_v, rel_v, idx_v, val_v):
    c = lax.axis_index("c")
    s = lax.axis_index("s")

    def _zinit(i, _):
        val_v[pl.ds(i * 16, 16)] = jnp.zeros((16,), jnp.float32)
        return 0
    lax.fori_loop(0, CHUNK // 16, _zinit, 0)
    zbase = s * ZSPAN
    nfull = ZSPAN // CHUNK
    tail = ZSPAN % CHUNK

    def _zcopy(t, _):
        pltpu.sync_copy(val_v, hist_sh.at[pl.ds(zbase + t * CHUNK, CHUNK)])
        return 0
    lax.fori_loop(0, nfull, _zcopy, 0)
    if tail:
        pltpu.sync_copy(val_v.at[pl.ds(0, tail)],
                        hist_sh.at[pl.ds(zbase + nfull * CHUNK, tail)])
    plsc.subcore_barrier()

    lo = c * HALF
    lanes = lax.iota(jnp.int32, 16)

    def _chunk(ch, _):
        base = pl.multiple_of(s * E_PER_TILE + ch * CHUNK, CHUNK)
        pltpu.sync_copy(dst_hbm.at[pl.ds(base, CHUNK)], dst_v)
        pltpu.sync_copy(rel_hbm.at[pl.ds(base, CHUNK)], rel_v)

        def _vec(j, _):
            off = j * 16
            d = dst_v[pl.ds(off, 16)]
            r = rel_v[pl.ds(off, 16)]
            local = d - lo
            ok = (local >= 0) & (local < HALF)
            flat = local * N_REL2 + r
            spread = ((r * 4096) + off + lanes) & 1048575
            idx_v[pl.ds(off, 16)] = jnp.where(ok, flat, spread)
            val_v[pl.ds(off, 16)] = jnp.where(
                ok, jnp.ones((16,), jnp.float32), jnp.zeros((16,), jnp.float32))
            return 0

        lax.fori_loop(0, CHUNK // 16, _vec, 0)
        pltpu.sync_copy(val_v, hist_sh.at[idx_v], add=True)
        return 0

    lax.fori_loop(0, N_CHUNKS, _chunk, 0)
    plsc.subcore_barrier()

    obase = c * HIST_WORDS + s * ZSPAN

    def _wb(t, _):
        pltpu.sync_copy(hist_sh.at[pl.ds(zbase + t * CHUNK, CHUNK)], val_v)
        pltpu.sync_copy(val_v, out_hbm.at[pl.ds(obase + t * CHUNK, CHUNK)])
        return 0
    lax.fori_loop(0, nfull, _wb, 0)
    if tail:
        toff = nfull * CHUNK
        pltpu.sync_copy(hist_sh.at[pl.ds(zbase + toff, tail)],
                        val_v.at[pl.ds(0, tail)])
        pltpu.sync_copy(val_v.at[pl.ds(0, tail)],
                        out_hbm.at[pl.ds(obase + toff, tail)])


@jax.jit
def _histogram(dst_pad, rel_pad):
    mesh = plsc.VectorSubcoreMesh(core_axis_name="c", subcore_axis_name="s")
    return pl.kernel(
        _hist_body,
        out_type=jax.ShapeDtypeStruct((N_ENT * N_REL2,), jnp.float32),
        mesh=mesh,
        scratch_types=[
            pltpu.VMEM_SHARED((HIST_WORDS,), jnp.float32),
            pltpu.VMEM((CHUNK,), jnp.int32),
            pltpu.VMEM((CHUNK,), jnp.int32),
            pltpu.VMEM((CHUNK,), jnp.int32),
            pltpu.VMEM((CHUNK,), jnp.float32),
        ],
    )(dst_pad, rel_pad)


BLK = 1000
GRID = N_ENT // BLK
_HI = lax.Precision.HIGHEST


def _dense_body(ent_ref, c_ref, relt_ref, rel_ref, nw_ref, h_ref, sum_ref, sq_ref):
    i = pl.program_id(0)
    ent = ent_ref[...]
    cnt = c_ref[...]
    s = jax.lax.dot_general(ent, relt_ref[...], (((1,), (0,)), ((), ())),
                            precision=_HI)
    mask = cnt > 0.5
    m = jnp.max(jnp.where(mask, s, -3.0e38), axis=1, keepdims=True)
    w = jnp.where(mask, cnt * jnp.exp(s - m), 0.0)
    denom = jnp.sum(w, axis=1, keepdims=True)
    numer = jax.lax.dot_general(w, rel_ref[...], (((1,), (0,)), ((), ())),
                                precision=_HI)
    neigh = numer * jnp.where(denom > 0.0, 1.0 / denom, 0.0)
    h = jax.lax.dot_general(neigh, nw_ref[...], (((1,), (0,)), ((), ())),
                            precision=_HI)
    h_ref[...] = h

    @pl.when(i == 0)
    def _init():
        sum_ref[...] = jnp.zeros_like(sum_ref)
        sq_ref[...] = jnp.zeros_like(sq_ref)

    sum_ref[...] += jnp.sum(h.reshape(BLK // 8, 8, EMB), axis=0)
    sq_ref[...] += jnp.sum((h * h).reshape(BLK // 8, 8, EMB), axis=0)


def _bn_body(h_ref, sum_ref, sq_ref, g_ref, b_ref, o_ref):
    tot = jnp.sum(sum_ref[...], axis=0, keepdims=True)
    tot2 = jnp.sum(sq_ref[...], axis=0, keepdims=True)
    mean = tot / N_ENT
    var = tot2 / N_ENT - mean * mean
    inv = jax.lax.rsqrt(var + 1e-5)
    o_ref[...] = jnp.tanh((h_ref[...] - mean) * inv * g_ref[...] + b_ref[...])


@jax.jit
def _dense(ent_emb, C, rel_t, rel_emb, neigh_w, gamma, beta):
    h, ssum, ssq = pl.pallas_call(
        _dense_body,
        grid=(GRID,),
        in_specs=[
            pl.BlockSpec((BLK, EMB), lambda i: (i, 0)),
            pl.BlockSpec((BLK, N_REL2), lambda i: (i, 0)),
            pl.BlockSpec((EMB, N_REL2), lambda i: (0, 0)),
            pl.BlockSpec((N_REL2, EMB), lambda i: (0, 0)),
            pl.BlockSpec((EMB, EMB), lambda i: (0, 0)),
        ],
        out_specs=[
            pl.BlockSpec((BLK, EMB), lambda i: (i, 0)),
            pl.BlockSpec((8, EMB), lambda i: (0, 0)),
            pl.BlockSpec((8, EMB), lambda i: (0, 0)),
        ],
        out_shape=[
            jax.ShapeDtypeStruct((N_ENT, EMB), jnp.float32),
            jax.ShapeDtypeStruct((8, EMB), jnp.float32),
            jax.ShapeDtypeStruct((8, EMB), jnp.float32),
        ],
    )(ent_emb, C, rel_t, rel_emb, neigh_w)

    return pl.pallas_call(
        _bn_body,
        grid=(GRID,),
        in_specs=[
            pl.BlockSpec((BLK, EMB), lambda i: (i, 0)),
            pl.BlockSpec((8, EMB), lambda i: (0, 0)),
            pl.BlockSpec((8, EMB), lambda i: (0, 0)),
            pl.BlockSpec((1, EMB), lambda i: (0, 0)),
            pl.BlockSpec((1, EMB), lambda i: (0, 0)),
        ],
        out_specs=pl.BlockSpec((BLK, EMB), lambda i: (i, 0)),
        out_shape=jax.ShapeDtypeStruct((N_ENT, EMB), jnp.float32),
    )(h, ssum, ssq, gamma, beta)


def kernel(ent_emb, rel_emb, edge_index, rel_id, neigh_w, bn_gamma, bn_beta):
    dst = edge_index[1]
    pad = E_PAD - N_EDGES
    dst_p = jnp.concatenate([dst, jnp.full((pad,), -1, jnp.int32)])
    rel_p = jnp.concatenate(
        [rel_id, (jnp.arange(pad, dtype=jnp.int32) % N_REL2)])
    c_flat = _histogram(dst_p, rel_p)
    C = c_flat.reshape(N_ENT, N_REL2)
    return _dense(ent_emb, C, rel_emb.T, rel_emb, neigh_w,
                  bn_gamma.reshape(1, EMB), bn_beta.reshape(1, EMB))


# speedup vs baseline: 43.7449x; 1.2605x over previous
import functools

import jax
import jax.numpy as jnp
from jax import lax
from jax.experimental import pallas as pl
from jax.experimental.pallas import tpu as pltpu
from jax.experimental.pallas import tpu_sc as plsc

N_ENT = 10000
N_REL2 = 400
EMB = 128
N_EDGES = 320000

NUM_SC = 2
NUM_TILES = 16
HALF = N_ENT // NUM_SC
HIST_WORDS = HALF * N_REL2
ZSPAN = HIST_WORDS // NUM_TILES
E_PER_TILE = 20480
CHUNK = 512
N_CHUNKS = E_PER_TILE // CHUNK
N_PAIRS = N_CHUNKS // 2
E_PAD = E_PER_TILE * NUM_TILES
PIECE = 512
NPIECE = ZSPAN // PIECE
PTAIL = ZSPAN % PIECE


def _hist_body(dst_hbm, rel_hbm, out_hbm, hist_sh,
               dst_a, rel_a, dst_b, rel_b, idx_a, val_a, idx_b, val_b,
               wb_c, wb_d,
               zsem, isem_a, isem_b, ssem_a, ssem_b, wisem, wosem):
    c = lax.axis_index("c")
    s = lax.axis_index("s")
    zbase = s * ZSPAN
    lo = c * HALF
    lanes = lax.iota(jnp.int32, 16)

    def _zinit(i, _):
        val_a[pl.ds(i * 16, 16)] = jnp.zeros((16,), jnp.float32)
        return 0
    lax.fori_loop(0, PIECE // 16, _zinit, 0)

    def _zpiece(t):
        return hist_sh.at[pl.ds(zbase + t * PIECE, PIECE)]

    def _zfire(t, _):
        @pl.when(t >= 4)
        def _():
            pltpu.make_async_copy(val_a, _zpiece(t - 4), zsem).wait()
        pltpu.async_copy(val_a, _zpiece(t), zsem)
        return 0
    lax.fori_loop(0, NPIECE, _zfire, 0)
    for t in range(NPIECE - 4, NPIECE):
        pltpu.make_async_copy(val_a, _zpiece(t), zsem).wait()
    if PTAIL:
        pltpu.sync_copy(val_a.at[pl.ds(0, PTAIL)],
                        hist_sh.at[pl.ds(zbase + NPIECE * PIECE, PTAIL)])
    plsc.subcore_barrier()

    def _ebase(ch):
        return pl.multiple_of(s * E_PER_TILE + ch * CHUNK, CHUNK)

    def _fire_in(ch, dbuf, rbuf, sem):
        b = _ebase(ch)
        pltpu.async_copy(dst_hbm.at[pl.ds(b, CHUNK)], dbuf, sem)
        pltpu.async_copy(rel_hbm.at[pl.ds(b, CHUNK)], rbuf, sem)

    def _wait_in(ch, dbuf, rbuf, sem):
        b = _ebase(ch)
        pltpu.make_async_copy(dst_hbm.at[pl.ds(b, CHUNK)], dbuf, sem).wait()
        pltpu.make_async_copy(rel_hbm.at[pl.ds(b, CHUNK)], rbuf, sem).wait()

    def _compute(dbuf, rbuf, ibuf, vbuf):
        def _vec(j, _):
            off = j * 16
            d = dbuf[pl.ds(off, 16)]
            r = rbuf[pl.ds(off, 16)]
            local = d - lo
            ok = (local >= 0) & (local < HALF)
            flat = local * N_REL2 + r
            spread = ((r * 4096) + off + lanes) & 1048575
            ibuf[pl.ds(off, 16)] = jnp.where(ok, flat, spread)
            vbuf[pl.ds(off, 16)] = jnp.where(
                ok, jnp.ones((16,), jnp.float32), jnp.zeros((16,), jnp.float32))
            return 0
        lax.fori_loop(0, CHUNK // 16, _vec, 0)

    _fire_in(0, dst_a, rel_a, isem_a)

    def _pair(p, _):
        _fire_in(2 * p + 1, dst_b, rel_b, isem_b)
        _wait_in(2 * p, dst_a, rel_a, isem_a)

        @pl.when(p > 0)
        def _():
            pltpu.make_async_copy(val_a, hist_sh.at[idx_a], ssem_a).wait()
        _compute(dst_a, rel_a, idx_a, val_a)
        pltpu.async_copy(val_a, hist_sh.at[idx_a], ssem_a, add=True)

        @pl.when(p < N_PAIRS - 1)
        def _():
            _fire_in(2 * p + 2, dst_a, rel_a, isem_a)
        _wait_in(2 * p + 1, dst_b, rel_b, isem_b)

        @pl.when(p > 0)
        def _():
            pltpu.make_async_copy(val_b, hist_sh.at[idx_b], ssem_b).wait()
        _compute(dst_b, rel_b, idx_b, val_b)
        pltpu.async_copy(val_b, hist_sh.at[idx_b], ssem_b, add=True)
        return 0

    lax.fori_loop(0, N_PAIRS, _pair, 0)
    pltpu.make_async_copy(val_a, hist_sh.at[idx_a], ssem_a).wait()
    pltpu.make_async_copy(val_b, hist_sh.at[idx_b], ssem_b).wait()
    plsc.subcore_barrier()

    obase = c * HIST_WORDS + s * ZSPAN
    wbufs = [val_a, val_b, wb_c, wb_d]

    def _opiece(t):
        return out_hbm.at[pl.ds(obase + t * PIECE, PIECE)]

    def _wb_in_wait(t, x):
        pltpu.make_async_copy(_zpiece(t), wbufs[x], wisem).wait()

    def _wb_out_wait(t, x):
        pltpu.make_async_copy(wbufs[x], _opiece(t), wosem).wait()

    def _group(g, _):
        for x in range(4):
            t = g * 4 + x

            @pl.when(g >= 1)
            def _(t=t, x=x):
                _wb_out_wait(t - 4, x)
            pltpu.async_copy(_zpiece(t), wbufs[x], wisem)
            x2 = (x + 2) % 4
            if x >= 2:
                _wb_in_wait(t - 2, x2)
                pltpu.async_copy(wbufs[x2], _opiece(t - 2), wosem)
            else:
                @pl.when(g >= 1)
                def _(t=t, x2=x2):
                    _wb_in_wait(t - 2, x2)
                    pltpu.async_copy(wbufs[x2], _opiece(t - 2), wosem)
        return 0

    lax.fori_loop(0, NPIECE // 4, _group, 0)
    for t in (NPIECE - 2, NPIECE - 1):
        x = t % 4
        _wb_in_wait(t, x)
        pltpu.async_copy(wbufs[x], _opiece(t), wosem)
    for t in range(NPIECE - 4, NPIECE):
        _wb_out_wait(t, t % 4)
    if PTAIL:
        pltpu.sync_copy(hist_sh.at[pl.ds(zbase + NPIECE * PIECE, PTAIL)],
                        wb_c.at[pl.ds(0, PTAIL)])
        pltpu.sync_copy(wb_c.at[pl.ds(0, PTAIL)],
                        out_hbm.at[pl.ds(obase + NPIECE * PIECE, PTAIL)])


@jax.jit
def _histogram(dst_pad, rel_pad):
    mesh = plsc.VectorSubcoreMesh(core_axis_name="c", subcore_axis_name="s")
    return pl.kernel(
        _hist_body,
        out_type=jax.ShapeDtypeStruct((N_ENT * N_REL2,), jnp.float32),
        mesh=mesh,
        scratch_types=[
            pltpu.VMEM_SHARED((HIST_WORDS,), jnp.float32),
            pltpu.VMEM((CHUNK,), jnp.int32),
            pltpu.VMEM((CHUNK,), jnp.int32),
            pltpu.VMEM((CHUNK,), jnp.int32),
            pltpu.VMEM((CHUNK,), jnp.int32),
            pltpu.VMEM((CHUNK,), jnp.int32),
            pltpu.VMEM((CHUNK,), jnp.float32),
            pltpu.VMEM((CHUNK,), jnp.int32),
            pltpu.VMEM((CHUNK,), jnp.float32),
            pltpu.VMEM((PIECE,), jnp.float32),
            pltpu.VMEM((PIECE,), jnp.float32),
        ] + [pltpu.SemaphoreType.DMA] * 7,
    )(dst_pad, rel_pad)


BLK = 1000
GRID = N_ENT // BLK
_HI = lax.Precision.HIGHEST


def _dense_body(ent_ref, c_ref, relt_ref, rel_ref, nw_ref, h_ref, sum_ref, sq_ref):
    i = pl.program_id(0)
    ent = ent_ref[...]
    cnt = c_ref[...]
    s = jax.lax.dot_general(ent, relt_ref[...], (((1,), (0,)), ((), ())),
                            precision=_HI)
    m = jnp.max(s, axis=1, keepdims=True)
    w = cnt * jnp.exp(s - m)
    denom = jnp.sum(w, axis=1, keepdims=True)
    numer = jax.lax.dot_general(w, rel_ref[...], (((1,), (0,)), ((), ())),
                                precision=_HI)
    neigh = numer * jnp.where(denom > 0.0, 1.0 / denom, 0.0)
    h = jax.lax.dot_general(neigh, nw_ref[...], (((1,), (0,)), ((), ())),
                            precision=_HI)
    h_ref[...] = h

    @pl.when(i == 0)
    def _init():
        sum_ref[...] = jnp.zeros_like(sum_ref)
        sq_ref[...] = jnp.zeros_like(sq_ref)

    sum_ref[...] += jnp.sum(h.reshape(BLK // 8, 8, EMB), axis=0)
    sq_ref[...] += jnp.sum((h * h).reshape(BLK // 8, 8, EMB), axis=0)


def _bn_body(h_ref, sum_ref, sq_ref, g_ref, b_ref, o_ref):
    tot = jnp.sum(sum_ref[...], axis=0, keepdims=True)
    tot2 = jnp.sum(sq_ref[...], axis=0, keepdims=True)
    mean = tot / N_ENT
    var = tot2 / N_ENT - mean * mean
    inv = jax.lax.rsqrt(var + 1e-5)
    o_ref[...] = jnp.tanh((h_ref[...] - mean) * inv * g_ref[...] + b_ref[...])


@jax.jit
def _dense(ent_emb, C, rel_t, rel_emb, neigh_w, gamma, beta):
    h, ssum, ssq = pl.pallas_call(
        _dense_body,
        grid=(GRID,),
        in_specs=[
            pl.BlockSpec((BLK, EMB), lambda i: (i, 0)),
            pl.BlockSpec((BLK, N_REL2), lambda i: (i, 0)),
            pl.BlockSpec((EMB, N_REL2), lambda i: (0, 0)),
            pl.BlockSpec((N_REL2, EMB), lambda i: (0, 0)),
            pl.BlockSpec((EMB, EMB), lambda i: (0, 0)),
        ],
        out_specs=[
            pl.BlockSpec((BLK, EMB), lambda i: (i, 0)),
            pl.BlockSpec((8, EMB), lambda i: (0, 0)),
            pl.BlockSpec((8, EMB), lambda i: (0, 0)),
        ],
        out_shape=[
            jax.ShapeDtypeStruct((N_ENT, EMB), jnp.float32),
            jax.ShapeDtypeStruct((8, EMB), jnp.float32),
            jax.ShapeDtypeStruct((8, EMB), jnp.float32),
        ],
    )(ent_emb, C, rel_t, rel_emb, neigh_w)

    return pl.pallas_call(
        _bn_body,
        grid=(GRID,),
        in_specs=[
            pl.BlockSpec((BLK, EMB), lambda i: (i, 0)),
            pl.BlockSpec((8, EMB), lambda i: (0, 0)),
            pl.BlockSpec((8, EMB), lambda i: (0, 0)),
            pl.BlockSpec((1, EMB), lambda i: (0, 0)),
            pl.BlockSpec((1, EMB), lambda i: (0, 0)),
        ],
        out_specs=pl.BlockSpec((BLK, EMB), lambda i: (i, 0)),
        out_shape=jax.ShapeDtypeStruct((N_ENT, EMB), jnp.float32),
    )(h, ssum, ssq, gamma, beta)


def kernel(ent_emb, rel_emb, edge_index, rel_id, neigh_w, bn_gamma, bn_beta):
    dst = edge_index[1]
    pad = E_PAD - N_EDGES
    dst_p = jnp.concatenate([dst, jnp.full((pad,), -1, jnp.int32)])
    rel_p = jnp.concatenate(
        [rel_id, (jnp.arange(pad, dtype=jnp.int32) % N_REL2)])
    c_flat = _histogram(dst_p, rel_p)
    C = c_flat.reshape(N_ENT, N_REL2)
    return _dense(ent_emb, C, rel_emb.T, rel_emb, neigh_w,
                  bn_gamma.reshape(1, EMB), bn_beta.reshape(1, EMB))


# speedup vs baseline: 67.5097x; 1.5433x over previous
import functools

import jax
import jax.numpy as jnp
from jax import lax
from jax.experimental import pallas as pl
from jax.experimental.pallas import tpu as pltpu
from jax.experimental.pallas import tpu_sc as plsc

N_ENT = 10000
N_REL2 = 400
EMB = 128
N_EDGES = 320000

NUM_SC = 2
NUM_TILES = 16
HALF = N_ENT // NUM_SC
HIST_WORDS = HALF * N_REL2
ZSPAN = HIST_WORDS // NUM_TILES
CHUNK = 512
N_CHUNKS = N_EDGES // CHUNK
N_PAIRS = 19
PIECE = 512
NPIECE = ZSPAN // PIECE
PTAIL = ZSPAN % PIECE


def _hist_body(ei_hbm, rel_hbm, out_hbm, hist_sh,
               dst_a, rel_a, dst_b, rel_b, idx_a, val_a, idx_b, val_b,
               zsem, isem_a, isem_b, ssem_a, ssem_b, wisem, wosem):
    c = lax.axis_index("c")
    s = lax.axis_index("s")
    zbase = s * ZSPAN
    lo = c * HALF
    lanes = lax.iota(jnp.int32, 16)

    def _zinit(i, _):
        val_a[pl.ds(i * 16, 16)] = jnp.zeros((16,), jnp.int32)
        return 0
    lax.fori_loop(0, PIECE // 16, _zinit, 0)

    def _zpiece(t):
        return hist_sh.at[pl.ds(zbase + t * PIECE, PIECE)]

    def _zfire(t, _):
        @pl.when(t >= 4)
        def _():
            pltpu.make_async_copy(val_a, _zpiece(t - 4), zsem).wait()
        pltpu.async_copy(val_a, _zpiece(t), zsem)
        return 0
    lax.fori_loop(0, NPIECE, _zfire, 0)
    for t in range(NPIECE - 4, NPIECE):
        pltpu.make_async_copy(val_a, _zpiece(t), zsem).wait()
    pltpu.sync_copy(val_a.at[pl.ds(0, PTAIL)],
                    hist_sh.at[pl.ds(zbase + NPIECE * PIECE, PTAIL)])
    plsc.subcore_barrier()

    def _ebase(j):
        return pl.multiple_of((j * 16 + s) * CHUNK, CHUNK)

    def _fire_in(j, dbuf, rbuf, sem):
        b = _ebase(j)
        pltpu.async_copy(ei_hbm.at[:, pl.ds(b, CHUNK)], dbuf, sem)
        pltpu.async_copy(rel_hbm.at[pl.ds(b, CHUNK)], rbuf, sem)

    def _wait_in(j, dbuf, rbuf, sem):
        b = _ebase(j)
        pltpu.make_async_copy(ei_hbm.at[:, pl.ds(b, CHUNK)], dbuf, sem).wait()
        pltpu.make_async_copy(rel_hbm.at[pl.ds(b, CHUNK)], rbuf, sem).wait()

    def _compute(dbuf, rbuf, ibuf, vbuf):
        def _vec(i, _):
            off = i * 16
            d = dbuf[1, pl.ds(off, 16)]
            r = rbuf[pl.ds(off, 16)]
            local = d - lo
            ok = (local >= 0) & (local < HALF)
            flat = local * N_REL2 + r
            spread = ((r * 4096) + off + lanes) & 1048575
            ibuf[pl.ds(off, 16)] = jnp.where(ok, flat, spread)
            vbuf[pl.ds(off, 16)] = jnp.where(
                ok, jnp.ones((16,), jnp.int32), jnp.zeros((16,), jnp.int32))
            return 0
        lax.fori_loop(0, CHUNK // 16, _vec, 0)

    def _scat_wait(vbuf, ibuf, sem):
        pltpu.make_async_copy(vbuf, hist_sh.at[ibuf], sem).wait()

    _fire_in(0, dst_a, rel_a, isem_a)

    def _pair(p, _):
        _fire_in(2 * p + 1, dst_b, rel_b, isem_b)
        _wait_in(2 * p, dst_a, rel_a, isem_a)

        @pl.when(p > 0)
        def _():
            _scat_wait(val_a, idx_a, ssem_a)
        _compute(dst_a, rel_a, idx_a, val_a)
        pltpu.async_copy(val_a, hist_sh.at[idx_a], ssem_a, add=True)

        _fire_in(2 * p + 2, dst_a, rel_a, isem_a)
        _wait_in(2 * p + 1, dst_b, rel_b, isem_b)

        @pl.when(p > 0)
        def _():
            _scat_wait(val_b, idx_b, ssem_b)
        _compute(dst_b, rel_b, idx_b, val_b)
        pltpu.async_copy(val_b, hist_sh.at[idx_b], ssem_b, add=True)
        return 0

    lax.fori_loop(0, N_PAIRS, _pair, 0)

    _wait_in(38, dst_a, rel_a, isem_a)
    _scat_wait(val_a, idx_a, ssem_a)
    _compute(dst_a, rel_a, idx_a, val_a)
    pltpu.async_copy(val_a, hist_sh.at[idx_a], ssem_a, add=True)

    @pl.when(s == 0)
    def _tail40():
        _fire_in(39, dst_b, rel_b, isem_b)
        _wait_in(39, dst_b, rel_b, isem_b)
        _scat_wait(val_b, idx_b, ssem_b)
        _compute(dst_b, rel_b, idx_b, val_b)
        pltpu.async_copy(val_b, hist_sh.at[idx_b], ssem_b, add=True)

    _scat_wait(val_a, idx_a, ssem_a)
    _scat_wait(val_b, idx_b, ssem_b)
    plsc.subcore_barrier()

    obase = c * HIST_WORDS + s * ZSPAN
    wbufs = [val_a, val_b, idx_a, idx_b]

    def _opiece(t):
        return out_hbm.at[pl.ds(obase + t * PIECE, PIECE)]

    def _win(t, x):
        pltpu.make_async_copy(_zpiece(t), wbufs[x], wisem).wait()

    def _wout(t, x):
        pltpu.make_async_copy(wbufs[x], _opiece(t), wosem).wait()

    def _group(g, _):
        for x in range(4):
            t = g * 4 + x

            @pl.when(g >= 1)
            def _(t=t, x=x):
                _wout(t - 4, x)
            pltpu.async_copy(_zpiece(t), wbufs[x], wisem)
            x2 = (x + 2) % 4
            if x >= 2:
                _win(t - 2, x2)
                pltpu.async_copy(wbufs[x2], _opiece(t - 2), wosem)
            else:
                @pl.when(g >= 1)
                def _(t=t, x2=x2):
                    _win(t - 2, x2)
                    pltpu.async_copy(wbufs[x2], _opiece(t - 2), wosem)
        return 0

    lax.fori_loop(0, NPIECE // 4, _group, 0)
    for t in range(NPIECE - 2, NPIECE):
        _win(t, t % 4)
        pltpu.async_copy(wbufs[t % 4], _opiece(t), wosem)
    for t in range(NPIECE - 4, NPIECE):
        _wout(t, t % 4)
    pltpu.sync_copy(hist_sh.at[pl.ds(zbase + NPIECE * PIECE, PTAIL)],
                    val_a.at[pl.ds(0, PTAIL)])
    pltpu.sync_copy(val_a.at[pl.ds(0, PTAIL)],
                    out_hbm.at[pl.ds(obase + NPIECE * PIECE, PTAIL)])

@jax.jit
def _histogram(edge_index, rel_id):
    mesh = plsc.VectorSubcoreMesh(core_axis_name="c", subcore_axis_name="s")
    return pl.kernel(
        _hist_body,
        out_type=jax.ShapeDtypeStruct((N_ENT * N_REL2,), jnp.int32),
        mesh=mesh,
        scratch_types=[
            pltpu.VMEM_SHARED((HIST_WORDS,), jnp.int32),
            pltpu.VMEM((2, CHUNK), jnp.int32),
            pltpu.VMEM((CHUNK,), jnp.int32),
            pltpu.VMEM((2, CHUNK), jnp.int32),
            pltpu.VMEM((CHUNK,), jnp.int32),
            pltpu.VMEM((CHUNK,), jnp.int32),
            pltpu.VMEM((CHUNK,), jnp.int32),
            pltpu.VMEM((CHUNK,), jnp.int32),
            pltpu.VMEM((CHUNK,), jnp.int32),
        ] + [pltpu.SemaphoreType.DMA] * 7,
    )(edge_index, rel_id)


BLK = 1000
GRID = N_ENT // BLK
_HI = lax.Precision.HIGHEST


def _dense_body(ent_ref, c_ref, relt_ref, rel_ref, nw_ref, g_ref, b_ref,
                o_ref, h_scr, sum_scr, sq_scr):
    ph = pl.program_id(0)
    i = pl.program_id(1)

    @pl.when(ph == 0)
    def _compute():
        ent = ent_ref[...]
        cnt = c_ref[...].astype(jnp.float32)
        s = jax.lax.dot_general(ent, relt_ref[...], (((1,), (0,)), ((), ())),
                                precision=_HI)
        m = jnp.max(s, axis=1, keepdims=True)
        w = cnt * jnp.exp(s - m)
        denom = jnp.sum(w, axis=1, keepdims=True)
        numer = jax.lax.dot_general(w, rel_ref[...], (((1,), (0,)), ((), ())))
        neigh = numer * jnp.where(denom > 0.0, 1.0 / denom, 0.0)
        h = jax.lax.dot_general(neigh, nw_ref[...], (((1,), (0,)), ((), ())))
        h_scr[pl.ds(i * BLK, BLK), :] = h

        @pl.when(i == 0)
        def _init():
            sum_scr[...] = jnp.zeros_like(sum_scr)
            sq_scr[...] = jnp.zeros_like(sq_scr)

        sum_scr[...] += jnp.sum(h.reshape(BLK // 8, 8, EMB), axis=0)
        sq_scr[...] += jnp.sum((h * h).reshape(BLK // 8, 8, EMB), axis=0)

    @pl.when(ph == 1)
    def _normalize():
        tot = jnp.sum(sum_scr[...], axis=0, keepdims=True)
        tot2 = jnp.sum(sq_scr[...], axis=0, keepdims=True)
        mean = tot / N_ENT
        var = tot2 / N_ENT - mean * mean
        inv = jax.lax.rsqrt(var + 1e-5)
        h = h_scr[pl.ds(i * BLK, BLK), :]
        o_ref[...] = jnp.tanh((h - mean) * inv * g_ref[...] + b_ref[...])


@jax.jit
def _dense(ent_emb, C, rel_t, rel_emb, neigh_w, gamma, beta):
    return pl.pallas_call(
        _dense_body,
        grid=(2, GRID),
        in_specs=[
            pl.BlockSpec((BLK, EMB), lambda p, i: (i * (1 - p), 0)),
            pl.BlockSpec((BLK, N_REL2), lambda p, i: (i * (1 - p), 0)),
            pl.BlockSpec((EMB, N_REL2), lambda p, i: (0, 0)),
            pl.BlockSpec((N_REL2, EMB), lambda p, i: (0, 0)),
            pl.BlockSpec((EMB, EMB), lambda p, i: (0, 0)),
            pl.BlockSpec((1, EMB), lambda p, i: (0, 0)),
            pl.BlockSpec((1, EMB), lambda p, i: (0, 0)),
        ],
        out_specs=pl.BlockSpec((BLK, EMB), lambda p, i: (i, 0)),
        out_shape=jax.ShapeDtypeStruct((N_ENT, EMB), jnp.float32),
        scratch_shapes=[
            pltpu.VMEM((N_ENT, EMB), jnp.float32),
            pltpu.VMEM((8, EMB), jnp.float32),
            pltpu.VMEM((8, EMB), jnp.float32),
        ],
    )(ent_emb, C, rel_t, rel_emb, neigh_w, gamma, beta)


def kernel(ent_emb, rel_emb, edge_index, rel_id, neigh_w, bn_gamma, bn_beta):
    c_flat = _histogram(edge_index, rel_id)
    C = c_flat.reshape(N_ENT, N_REL2)
    return _dense(ent_emb, C, rel_emb.T, rel_emb, neigh_w,
                  bn_gamma.reshape(1, EMB), bn_beta.reshape(1, EMB))


# speedup vs baseline: 72.2240x; 1.0698x over previous
import functools

import jax
import jax.numpy as jnp
from jax import lax
from jax.experimental import pallas as pl
from jax.experimental.pallas import tpu as pltpu
from jax.experimental.pallas import tpu_sc as plsc

N_ENT = 10000
N_REL2 = 400
EMB = 128
N_EDGES = 320000

NUM_SC = 2
NUM_TILES = 16
HALF = N_ENT // NUM_SC
HIST_WORDS = HALF * N_REL2
ZSPAN = HIST_WORDS // NUM_TILES
CHUNK = 512
N_CHUNKS = N_EDGES // CHUNK
N_PAIRS = 19
PIECE = 512
NPIECE = ZSPAN // PIECE
PTAIL = ZSPAN % PIECE


def _hist_body(ei_hbm, rel_hbm, out_hbm, hist_sh,
               dst_a, rel_a, dst_b, rel_b, idx_a, val_a, idx_b, val_b,
               zsem, isem_a, isem_b, ssem_a, ssem_b, wisem, wosem):
    c = lax.axis_index("c")
    s = lax.axis_index("s")
    zbase = s * ZSPAN
    lo = c * HALF
    lanes = lax.iota(jnp.int32, 16)

    def _zinit(i, _):
        val_a[pl.ds(i * 16, 16)] = jnp.zeros((16,), jnp.int32)
        return 0
    lax.fori_loop(0, PIECE // 16, _zinit, 0)

    def _zpiece(t):
        return hist_sh.at[pl.ds(zbase + t * PIECE, PIECE)]

    def _zfire(t, _):
        @pl.when(t >= 4)
        def _():
            pltpu.make_async_copy(val_a, _zpiece(t - 4), zsem).wait()
        pltpu.async_copy(val_a, _zpiece(t), zsem)
        return 0
    lax.fori_loop(0, NPIECE, _zfire, 0)
    for t in range(NPIECE - 4, NPIECE):
        pltpu.make_async_copy(val_a, _zpiece(t), zsem).wait()
    pltpu.sync_copy(val_a.at[pl.ds(0, PTAIL)],
                    hist_sh.at[pl.ds(zbase + NPIECE * PIECE, PTAIL)])
    plsc.subcore_barrier()

    def _ebase(j):
        return pl.multiple_of((j * 16 + s) * CHUNK, CHUNK)

    def _fire_in(j, dbuf, rbuf, sem):
        b = _ebase(j)
        pltpu.async_copy(ei_hbm.at[:, pl.ds(b, CHUNK)], dbuf, sem)
        pltpu.async_copy(rel_hbm.at[pl.ds(b, CHUNK)], rbuf, sem)

    def _wait_in(j, dbuf, rbuf, sem):
        b = _ebase(j)
        pltpu.make_async_copy(ei_hbm.at[:, pl.ds(b, CHUNK)], dbuf, sem).wait()
        pltpu.make_async_copy(rel_hbm.at[pl.ds(b, CHUNK)], rbuf, sem).wait()

    def _compute(dbuf, rbuf, ibuf, vbuf):
        def _vec(i, _):
            off = i * 16
            d = dbuf[1, pl.ds(off, 16)]
            r = rbuf[pl.ds(off, 16)]
            local = d - lo
            ok = (local >= 0) & (local < HALF)
            flat = local * N_REL2 + r
            spread = ((r * 4096) + off + lanes) & 1048575
            ibuf[pl.ds(off, 16)] = jnp.where(ok, flat, spread)
            vbuf[pl.ds(off, 16)] = jnp.where(
                ok, jnp.ones((16,), jnp.int32), jnp.zeros((16,), jnp.int32))
            return 0
        lax.fori_loop(0, CHUNK // 16, _vec, 0)

    def _scat_wait(vbuf, ibuf, sem):
        pltpu.make_async_copy(vbuf, hist_sh.at[ibuf], sem).wait()

    _fire_in(0, dst_a, rel_a, isem_a)

    def _pair(p, _):
        _fire_in(2 * p + 1, dst_b, rel_b, isem_b)
        _wait_in(2 * p, dst_a, rel_a, isem_a)

        @pl.when(p > 0)
        def _():
            _scat_wait(val_a, idx_a, ssem_a)
        _compute(dst_a, rel_a, idx_a, val_a)
        pltpu.async_copy(val_a, hist_sh.at[idx_a], ssem_a, add=True)

        _fire_in(2 * p + 2, dst_a, rel_a, isem_a)
        _wait_in(2 * p + 1, dst_b, rel_b, isem_b)

        @pl.when(p > 0)
        def _():
            _scat_wait(val_b, idx_b, ssem_b)
        _compute(dst_b, rel_b, idx_b, val_b)
        pltpu.async_copy(val_b, hist_sh.at[idx_b], ssem_b, add=True)
        return 0

    lax.fori_loop(0, N_PAIRS, _pair, 0)

    _wait_in(38, dst_a, rel_a, isem_a)
    _scat_wait(val_a, idx_a, ssem_a)
    _compute(dst_a, rel_a, idx_a, val_a)
    pltpu.async_copy(val_a, hist_sh.at[idx_a], ssem_a, add=True)

    @pl.when(s == 0)
    def _tail40():
        _fire_in(39, dst_b, rel_b, isem_b)
        _wait_in(39, dst_b, rel_b, isem_b)
        _scat_wait(val_b, idx_b, ssem_b)
        _compute(dst_b, rel_b, idx_b, val_b)
        pltpu.async_copy(val_b, hist_sh.at[idx_b], ssem_b, add=True)

    _scat_wait(val_a, idx_a, ssem_a)
    _scat_wait(val_b, idx_b, ssem_b)
    plsc.subcore_barrier()

    obase = c * HIST_WORDS + s * ZSPAN
    wbufs = [val_a, val_b, idx_a, idx_b]

    def _opiece(t):
        return out_hbm.at[pl.ds(obase + t * PIECE, PIECE)]

    def _win(t, x):
        pltpu.make_async_copy(_zpiece(t), wbufs[x], wisem).wait()

    def _wout(t, x):
        pltpu.make_async_copy(wbufs[x], _opiece(t), wosem).wait()

    def _group(g, _):
        for x in range(4):
            t = g * 4 + x

            @pl.when(g >= 1)
            def _(t=t, x=x):
                _wout(t - 4, x)
            pltpu.async_copy(_zpiece(t), wbufs[x], wisem)
            x2 = (x + 2) % 4
            if x >= 2:
                _win(t - 2, x2)
                pltpu.async_copy(wbufs[x2], _opiece(t - 2), wosem)
            else:
                @pl.when(g >= 1)
                def _(t=t, x2=x2):
                    _win(t - 2, x2)
                    pltpu.async_copy(wbufs[x2], _opiece(t - 2), wosem)
        return 0

    lax.fori_loop(0, NPIECE // 4, _group, 0)
    for t in range(NPIECE - 2, NPIECE):
        _win(t, t % 4)
        pltpu.async_copy(wbufs[t % 4], _opiece(t), wosem)
    for t in range(NPIECE - 4, NPIECE):
        _wout(t, t % 4)
    pltpu.sync_copy(hist_sh.at[pl.ds(zbase + NPIECE * PIECE, PTAIL)],
                    val_a.at[pl.ds(0, PTAIL)])
    pltpu.sync_copy(val_a.at[pl.ds(0, PTAIL)],
                    out_hbm.at[pl.ds(obase + NPIECE * PIECE, PTAIL)])

@jax.jit
def _histogram(edge_index, rel_id):
    mesh = plsc.VectorSubcoreMesh(core_axis_name="c", subcore_axis_name="s")
    return pl.kernel(
        _hist_body,
        out_type=jax.ShapeDtypeStruct((N_ENT * N_REL2,), jnp.int32),
        mesh=mesh,
        scratch_types=[
            pltpu.VMEM_SHARED((HIST_WORDS,), jnp.int32),
            pltpu.VMEM((2, CHUNK), jnp.int32),
            pltpu.VMEM((CHUNK,), jnp.int32),
            pltpu.VMEM((2, CHUNK), jnp.int32),
            pltpu.VMEM((CHUNK,), jnp.int32),
            pltpu.VMEM((CHUNK,), jnp.int32),
            pltpu.VMEM((CHUNK,), jnp.int32),
            pltpu.VMEM((CHUNK,), jnp.int32),
            pltpu.VMEM((CHUNK,), jnp.int32),
        ] + [pltpu.SemaphoreType.DMA] * 7,
    )(edge_index, rel_id)


BLK = 1000
GRID = N_ENT // BLK
_HI = lax.Precision.HIGHEST


def _score_body(ent_ref, relt_ref, s_ref):
    s_ref[...] = jax.lax.dot_general(
        ent_ref[...], relt_ref[...], (((1,), (0,)), ((), ())), precision=_HI)


@jax.jit
def _scores(ent_emb, rel_t):
    return pl.pallas_call(
        _score_body,
        grid=(GRID,),
        in_specs=[
            pl.BlockSpec((BLK, EMB), lambda i: (i, 0)),
            pl.BlockSpec((EMB, N_REL2), lambda i: (0, 0)),
        ],
        out_specs=pl.BlockSpec((BLK, N_REL2), lambda i: (i, 0)),
        out_shape=jax.ShapeDtypeStruct((N_ENT, N_REL2), jnp.float32),
    )(ent_emb, rel_t)


def _dense_body(s_ref, c_ref, rel_ref, nw_ref, g_ref, b_ref,
                o_ref, h_scr, sum_scr, sq_scr):
    ph = pl.program_id(0)
    i = pl.program_id(1)

    @pl.when(ph == 0)
    def _compute():
        cnt = c_ref[...].astype(jnp.float32)
        s = s_ref[...]
        m = jnp.max(s, axis=1, keepdims=True)
        w = cnt * jnp.exp(s - m)
        denom = jnp.sum(w, axis=1, keepdims=True)
        numer = jax.lax.dot_general(w, rel_ref[...], (((1,), (0,)), ((), ())))
        neigh = numer * jnp.where(denom > 0.0, 1.0 / denom, 0.0)
        h = jax.lax.dot_general(neigh, nw_ref[...], (((1,), (0,)), ((), ())))
        h_scr[pl.ds(i * BLK, BLK), :] = h

        @pl.when(i == 0)
        def _init():
            sum_scr[...] = jnp.zeros_like(sum_scr)
            sq_scr[...] = jnp.zeros_like(sq_scr)

        sum_scr[...] += jnp.sum(h.reshape(BLK // 8, 8, EMB), axis=0)
        sq_scr[...] += jnp.sum((h * h).reshape(BLK // 8, 8, EMB), axis=0)

    @pl.when(ph == 1)
    def _normalize():
        tot = jnp.sum(sum_scr[...], axis=0, keepdims=True)
        tot2 = jnp.sum(sq_scr[...], axis=0, keepdims=True)
        mean = tot / N_ENT
        var = tot2 / N_ENT - mean * mean
        inv = jax.lax.rsqrt(var + 1e-5)
        h = h_scr[pl.ds(i * BLK, BLK), :]
        o_ref[...] = jnp.tanh((h - mean) * inv * g_ref[...] + b_ref[...])


@jax.jit
def _dense(S, C, rel_emb, neigh_w, gamma, beta):
    return pl.pallas_call(
        _dense_body,
        grid=(2, GRID),
        in_specs=[
            pl.BlockSpec((BLK, N_REL2), lambda p, i: (i * (1 - p), 0)),
            pl.BlockSpec((BLK, N_REL2), lambda p, i: (i * (1 - p), 0)),
            pl.BlockSpec((N_REL2, EMB), lambda p, i: (0, 0)),
            pl.BlockSpec((EMB, EMB), lambda p, i: (0, 0)),
            pl.BlockSpec((1, EMB), lambda p, i: (0, 0)),
            pl.BlockSpec((1, EMB), lambda p, i: (0, 0)),
        ],
        out_specs=pl.BlockSpec((BLK, EMB), lambda p, i: (i, 0)),
        out_shape=jax.ShapeDtypeStruct((N_ENT, EMB), jnp.float32),
        scratch_shapes=[
            pltpu.VMEM((N_ENT, EMB), jnp.float32),
            pltpu.VMEM((8, EMB), jnp.float32),
            pltpu.VMEM((8, EMB), jnp.float32),
        ],
    )(S, C, rel_emb, neigh_w, gamma, beta)


def kernel(ent_emb, rel_emb, edge_index, rel_id, neigh_w, bn_gamma, bn_beta):
    S = _scores(ent_emb, rel_emb.T)
    c_flat = _histogram(edge_index, rel_id)
    C = c_flat.reshape(N_ENT, N_REL2)
    return _dense(S, C, rel_emb, neigh_w,
                  bn_gamma.reshape(1, EMB), bn_beta.reshape(1, EMB))
